# Initial kernel scaffold; baseline (speedup 1.0000x reference)
import jax
import jax.numpy as jnp
from jax.experimental import pallas as pl


def kernel(pos, c_noise, params, edge_index, bond_mask, atom_types):
    raise NotImplementedError("write your pallas kernel here")


# speedup vs baseline: 1.0476x; 1.0476x over previous
import functools
import math

import jax
import jax.numpy as jnp
from jax.experimental import pallas as pl

_N = 50000
_E = 800000
_RAD = 8
_CUTOFF = 5.0
_AVG_DEG = 16.0

_EDGE_BLK = 2000


def _edge_dense_body(ev_ref, bm_ref, w_ref, out_sh_ref, *rad_refs):
    ev = ev_ref[...]
    bm = bm_ref[...]
    r2 = jnp.sum(ev * ev, axis=1, keepdims=True)
    r = jnp.sqrt(r2)
    u = ev / jnp.maximum(r, 1e-9)
    sh = jnp.concatenate([jnp.ones_like(r), jnp.sqrt(3.0) * u], axis=1)
    out_sh_ref[...] = sh

    step = _CUTOFF / (_RAD + 1)
    k_iota = jax.lax.broadcasted_iota(jnp.int32, (ev.shape[0], _RAD), 1)
    diff = r / step - (k_iota.astype(jnp.float32) + 1.0)
    rad_attr = jnp.exp(-diff * diff) / 1.12

    emb_bond = w_ref[0:2, 0:8]
    bonded = jnp.where(bm > 0, emb_bond[1][None, :], emb_bond[0][None, :])

    for i in range(3):
        base = 2 + i * (16 + 1 + 64)
        w1 = w_ref[base : base + 16, 0:64]
        b1 = w_ref[base + 16, 0:64]
        w2 = w_ref[base + 17 : base + 81, 0:64]
        h = (
            jnp.dot(bonded, w1[0:8], preferred_element_type=jnp.float32)
            + jnp.dot(rad_attr, w1[8:16], preferred_element_type=jnp.float32)
            + b1[None, :]
        )
        h = jnp.maximum(h, 0.0)
        rad_refs[i][...] = jnp.dot(h, w2, preferred_element_type=jnp.float32)


def _pack_edge_weights(params):
    rows = [jnp.pad(params["emb_bond"], ((0, 0), (0, 56)))]
    for i in range(3):
        p = params["conv%d" % i]
        rows.append(p["W1"])
        rows.append(p["b1"][None, :])
        rows.append(p["W2"])
    return jnp.concatenate(rows, axis=0)


@jax.jit
def _edge_dense(edge_vec, bond_mask, wpack):
    nblk = _E // _EDGE_BLK
    out_shapes = [jax.ShapeDtypeStruct((_E, 4), jnp.float32)] + [
        jax.ShapeDtypeStruct((_E, 64), jnp.float32) for _ in range(3)
    ]
    grid = (nblk,)
    in_specs = [
        pl.BlockSpec((_EDGE_BLK, 3), lambda i: (i, 0)),
        pl.BlockSpec((_EDGE_BLK, 1), lambda i: (i, 0)),
        pl.BlockSpec(wpack.shape, lambda i: (0, 0)),
    ]
    out_specs = [pl.BlockSpec((_EDGE_BLK, 4), lambda i: (i, 0))] + [
        pl.BlockSpec((_EDGE_BLK, 64), lambda i: (i, 0)) for _ in range(3)
    ]
    return pl.pallas_call(
        _edge_dense_body,
        grid=grid,
        in_specs=in_specs,
        out_specs=out_specs,
        out_shape=out_shapes,
    )(edge_vec, bond_mask.astype(jnp.int32)[:, None], wpack)


def kernel(pos, c_noise, params, edge_index, bond_mask, atom_types):
    src = edge_index[0]
    dst = edge_index[1]
    inv_sqrt_deg = 1.0 / math.sqrt(_AVG_DEG)

    edge_vec = pos[src] - pos[dst]
    wpack = _pack_edge_weights(params)
    sh, rad0, rad1, rad2 = _edge_dense(edge_vec, bond_mask, wpack)
    radials = (rad0, rad1, rad2)

    S = jax.ops.segment_sum(sh, dst, num_segments=_N)

    x = params["emb_atom"][atom_types]
    x = x * (c_noise[:, None] * params["scale0"]["g"] + params["scale0"]["b"])

    def conv(xs, i):
        p = params["conv%d" % i]
        y = xs @ p["Wself"]
        agg = jax.ops.segment_sum(y[src] * radials[i], dst, num_segments=_N)
        return (agg + S @ p["Wsh"]) * inv_sqrt_deg

    x = conv(x, 0)
    for i in (1, 2):
        sc = params["scale%d" % i]
        xs = x * (c_noise[:, None] * sc["g"] + sc["b"])
        xn = conv(xs, i)
        sk = params["skip%d" % i]
        alpha = jax.nn.sigmoid(c_noise * sk["w"] + sk["b"])[:, None]
        x = alpha * x + xn
    return (x @ params["Wout"] + params["bout"]) * params["gain"]


# speedup vs baseline: 2.0621x; 1.9684x over previous
import functools
import math

import jax
import jax.numpy as jnp
from jax import lax
from jax.experimental import pallas as pl
from jax.experimental.pallas import tpu as pltpu
from jax.experimental.pallas import tpu_sc as plsc

_N = 50000
_E = 800000
_RAD = 8
_CUTOFF = 5.0
_AVG_DEG = 16.0
_INV_SQRT_DEG = 1.0 / math.sqrt(_AVG_DEG)

_NC = 2
_NS = 16
_W = 128
_ER = 6400
_EP = _ER * _W
_RC = 8
_NROW = _N // _NS

_EDGE_BLK = 2048
_NODE_BLK = 2000

_mesh = plsc.VectorSubcoreMesh(core_axis_name="c", subcore_axis_name="s")
_sc_params = pltpu.CompilerParams(use_tc_tiling_on_sc=False)


@functools.partial(
    pl.kernel,
    out_type=jax.ShapeDtypeStruct((_ER, _W, 16), jnp.float32),
    mesh=_mesh,
    compiler_params=_sc_params,
    scratch_types=[
        pltpu.VMEM((_RC, _W), jnp.int32),
        pltpu.VMEM((_RC, _W, 16), jnp.float32),
        pltpu.VMEM((_RC, _W, 16), jnp.float32),
        pltpu.SemaphoreType.DMA,
        pltpu.SemaphoreType.DMA,
    ],
)
def _sc_edge_vec(pos16, srcs4, dst3, out_ev, idx_v, rs_v, rd_v, sem, sem2):
    c = lax.axis_index("c")
    s = lax.axis_index("s")
    wid = s * _NC + c
    rows_per_w = _ER // (_NC * _NS)
    nchunk = rows_per_w // _RC

    def chunk(k, carry):
        row0 = wid * rows_per_w + k * _RC
        pltpu.sync_copy(srcs4.at[0, pl.ds(row0, _RC)], idx_v)
        cps = [
            pltpu.async_copy(pos16.at[idx_v.at[j]], rs_v.at[j], sem)
            for j in range(_RC)
        ]
        for cp in cps:
            cp.wait()
        pltpu.sync_copy(dst3.at[pl.ds(row0, _RC)], idx_v)
        cps = [
            pltpu.async_copy(pos16.at[idx_v.at[j]], rd_v.at[j], sem2)
            for j in range(_RC)
        ]
        for cp in cps:
            cp.wait()
        for j in range(_RC):

            @plsc.parallel_loop(0, _W, unroll=8)
            def _sub(i):
                rs_v[j, i, :] = rs_v[j, i, :] - rd_v[j, i, :]

        pltpu.sync_copy(rs_v, out_ev.at[pl.ds(row0, _RC)])
        return carry

    lax.fori_loop(0, nchunk, chunk, 0)


@functools.partial(
    pl.kernel,
    out_type=jax.ShapeDtypeStruct((_NC, _N, 16), jnp.float32),
    mesh=_mesh,
    compiler_params=_sc_params,
    scratch_types=[
        pltpu.VMEM((_RC, _W), jnp.int32),
        pltpu.VMEM((_RC, _W, 16), jnp.float32),
        pltpu.VMEM_SHARED((_N, 16), jnp.float32),
        pltpu.SemaphoreType.DMA,
    ],
)
def _sc_scatter_sh(sh3, dst3, zeros16, out, idx_v, sh_v, acc, sem):
    c = lax.axis_index("c")
    s = lax.axis_index("s")
    wid = s * _NC + c
    rows_per_w = _ER // (_NC * _NS)
    nchunk = rows_per_w // _RC

    pltpu.sync_copy(zeros16.at[pl.ds(s * _NROW, _NROW)],
                    acc.at[pl.ds(s * _NROW, _NROW)])
    plsc.subcore_barrier()

    def chunk(k, carry):
        row0 = wid * rows_per_w + k * _RC
        pltpu.sync_copy(dst3.at[pl.ds(row0, _RC)], idx_v)
        pltpu.sync_copy(sh3.at[pl.ds(row0, _RC)], sh_v)
        for j in range(_RC):
            pltpu.sync_copy(sh_v.at[j], acc.at[idx_v.at[j]], add=True)
        return carry

    lax.fori_loop(0, nchunk, chunk, 0)
    plsc.subcore_barrier()
    pltpu.sync_copy(acc.at[pl.ds(s * _NROW, _NROW)],
                    out.at[c, pl.ds(s * _NROW, _NROW)])


@functools.partial(
    pl.kernel,
    out_type=jax.ShapeDtypeStruct((4, _N, 16), jnp.float32),
    mesh=_mesh,
    compiler_params=_sc_params,
    scratch_types=[
        pltpu.VMEM((_RC, _W), jnp.int32),
        pltpu.VMEM((_RC, _W), jnp.int32),
        pltpu.VMEM((_RC, _W, 16), jnp.float32),
        pltpu.VMEM((2, _W, 16), jnp.float32),
        pltpu.VMEM_SHARED((_N, 16), jnp.float32),
        pltpu.SemaphoreType.DMA,
        pltpu.SemaphoreType.DMA,
        pltpu.SemaphoreType.DMA,
        pltpu.SemaphoreType.DMA,
    ],
)
def _sc_conv(y4, rad4, srcs4, dst3, zeros16, out,
             idx_s, idx_d, rad_v, rows_v, acc, g0, g1, s0, s1):
    c = lax.axis_index("c")
    s = lax.axis_index("s")
    rows_per_s = _ER // _NS
    nchunk = rows_per_s // _RC
    gsem = (g0, g1)
    ssem = (s0, s1)

    for p in range(2):
        q = 2 * c + p
        pltpu.sync_copy(zeros16.at[pl.ds(s * _NROW, _NROW)],
                        acc.at[pl.ds(s * _NROW, _NROW)])
        plsc.subcore_barrier()

        def chunk(k, carry):
            row0 = s * rows_per_s + k * _RC
            pltpu.sync_copy(srcs4.at[q, pl.ds(row0, _RC)], idx_s)
            pltpu.sync_copy(dst3.at[pl.ds(row0, _RC)], idx_d)
            pltpu.sync_copy(rad4.at[q, pl.ds(row0, _RC)], rad_v)
            gcp = [None] * _RC
            scp = [None] * _RC
            gcp[0] = pltpu.async_copy(y4.at[idx_s.at[0]], rows_v.at[0],
                                      gsem[0])
            for j in range(_RC):
                slot = j % 2
                nslot = (j + 1) % 2
                if j + 1 < _RC:
                    if j >= 1:
                        scp[j - 1].wait()
                    gcp[j + 1] = pltpu.async_copy(
                        y4.at[idx_s.at[j + 1]], rows_v.at[nslot],
                        gsem[nslot])
                gcp[j].wait()

                @plsc.parallel_loop(0, _W, unroll=8)
                def _mul(i):
                    rows_v[slot, i, :] = rows_v[slot, i, :] * rad_v[j, i, :]

                scp[j] = pltpu.async_copy(rows_v.at[slot],
                                          acc.at[idx_d.at[j]], ssem[slot],
                                          add=True)
            scp[_RC - 2].wait()
            scp[_RC - 1].wait()
            return carry

        lax.fori_loop(0, nchunk, chunk, 0)
        plsc.subcore_barrier()
        pltpu.sync_copy(acc.at[pl.ds(s * _NROW, _NROW)],
                        out.at[q, pl.ds(s * _NROW, _NROW)])
        if p == 0:
            plsc.subcore_barrier()


def _edge_dense_body(ev_ref, bm_ref, w_ref, out_sh_ref, *rad_refs):
    ev = ev_ref[...]
    bm = bm_ref[...]
    B = ev.shape[0]
    blk = pl.program_id(0)
    row_iota = jax.lax.broadcasted_iota(jnp.int32, (B, 1), 0)
    valid = (blk * B + row_iota) < _E

    r2 = jnp.sum(ev * ev, axis=1, keepdims=True)
    r = jnp.sqrt(r2)
    u = ev[:, 0:3] / jnp.maximum(r, 1e-9)
    vmask = valid.astype(jnp.float32)
    sh = jnp.concatenate(
        [vmask, jnp.sqrt(3.0) * u * vmask,
         jnp.zeros((B, 12), jnp.float32)], axis=1
    )
    out_sh_ref[...] = sh

    step = _CUTOFF / (_RAD + 1)
    k_iota = jax.lax.broadcasted_iota(jnp.int32, (B, _RAD), 1)
    diff = r / step - (k_iota.astype(jnp.float32) + 1.0)
    rad_attr = jnp.exp(-diff * diff) / 1.12

    emb_bond = w_ref[0:2, 0:8]
    bonded = jnp.where(bm > 0, emb_bond[1][None, :], emb_bond[0][None, :])

    for i in range(3):
        base = 2 + i * (16 + 1 + 64)
        w1 = w_ref[base : base + 16, 0:64]
        b1 = w_ref[base + 16, 0:64]
        w2 = w_ref[base + 17 : base + 81, 0:64]
        h = (
            jnp.dot(bonded, w1[0:8], preferred_element_type=jnp.float32)
            + jnp.dot(rad_attr, w1[8:16], preferred_element_type=jnp.float32)
            + b1[None, :]
        )
        h = jnp.maximum(h, 0.0)
        radial = jnp.dot(h, w2, preferred_element_type=jnp.float32) * vmask
        for qq in range(4):
            rad_refs[i][qq] = radial[:, 16 * qq : 16 * qq + 16]


def _pack_edge_weights(params):
    rows = [jnp.pad(params["emb_bond"], ((0, 0), (0, 56)))]
    for i in range(3):
        p = params["conv%d" % i]
        rows.append(p["W1"])
        rows.append(p["b1"][None, :])
        rows.append(p["W2"])
    return jnp.concatenate(rows, axis=0)


def _edge_dense(ev, bm_pad, wpack):
    nblk = _EP // _EDGE_BLK
    out_shapes = [jax.ShapeDtypeStruct((_EP, 16), jnp.float32)] + [
        jax.ShapeDtypeStruct((4, _EP, 16), jnp.float32) for _ in range(3)
    ]
    in_specs = [
        pl.BlockSpec((_EDGE_BLK, 16), lambda i: (i, 0)),
        pl.BlockSpec((_EDGE_BLK, 1), lambda i: (i, 0)),
        pl.BlockSpec(wpack.shape, lambda i: (0, 0)),
    ]
    out_specs = [pl.BlockSpec((_EDGE_BLK, 16), lambda i: (i, 0))] + [
        pl.BlockSpec((4, _EDGE_BLK, 16), lambda i: (0, i, 0))
        for _ in range(3)
    ]
    return pl.pallas_call(
        _edge_dense_body,
        grid=(nblk,),
        in_specs=in_specs,
        out_specs=out_specs,
        out_shape=out_shapes,
    )(ev, bm_pad, wpack)


def _node_first_body(x_ref, cn_ref, g_ref, b_ref, ws_ref, y_ref):
    cn = cn_ref[...]
    xs = x_ref[...] * (cn * g_ref[...] + b_ref[...])
    y = jnp.dot(xs, ws_ref[...], preferred_element_type=jnp.float32)
    for qq in range(4):
        y_ref[qq] = y[:, 16 * qq : 16 * qq + 16]


def _node_mid_body(agg_ref, s_ref, xp_ref, cn_ref, wsh_ref, sw_ref, sb_ref,
                   g_ref, b_ref, ws_ref, y_ref, x_ref, *, first):
    cn = cn_ref[...]
    xn = jnp.concatenate(
        [agg_ref[0], agg_ref[1], agg_ref[2], agg_ref[3]], axis=1)
    xn = (xn + jnp.dot(s_ref[...], wsh_ref[...],
                       preferred_element_type=jnp.float32)) * _INV_SQRT_DEG
    if first:
        x = xn
    else:
        alpha = jax.nn.sigmoid(cn * sw_ref[0, 0] + sb_ref[0, 0])
        x = alpha * xp_ref[...] + xn
    x_ref[...] = x
    xs = x * (cn * g_ref[...] + b_ref[...])
    y = jnp.dot(xs, ws_ref[...], preferred_element_type=jnp.float32)
    for qq in range(4):
        y_ref[qq] = y[:, 16 * qq : 16 * qq + 16]


def _node_last_body(agg_ref, s_ref, xp_ref, cn_ref, wsh_ref, sw_ref, sb_ref,
                    wout_ref, bout_ref, gain_ref, out_ref):
    cn = cn_ref[...]
    xn = jnp.concatenate(
        [agg_ref[0], agg_ref[1], agg_ref[2], agg_ref[3]], axis=1)
    xn = (xn + jnp.dot(s_ref[...], wsh_ref[...],
                       preferred_element_type=jnp.float32)) * _INV_SQRT_DEG
    alpha = jax.nn.sigmoid(cn * sw_ref[0, 0] + sb_ref[0, 0])
    x = alpha * xp_ref[...] + xn
    out_ref[...] = (
        jnp.dot(x, wout_ref[...], preferred_element_type=jnp.float32)
        + bout_ref[...]
    ) * gain_ref[0, 0]


def _full(shape):
    return pl.BlockSpec(shape, lambda i: tuple(0 for _ in shape))


_BS_N64 = pl.BlockSpec((_NODE_BLK, 64), lambda i: (i, 0))
_BS_N1 = pl.BlockSpec((_NODE_BLK, 1), lambda i: (i, 0))
_BS_N16 = pl.BlockSpec((_NODE_BLK, 16), lambda i: (i, 0))
_BS_AGG = pl.BlockSpec((4, _NODE_BLK, 16), lambda i: (0, i, 0))
_Y_SHAPE = jax.ShapeDtypeStruct((4, _N, 16), jnp.float32)
_X_SHAPE = jax.ShapeDtypeStruct((_N, 64), jnp.float32)


def _node_first(x_emb, cn, g, b, ws):
    return pl.pallas_call(
        _node_first_body,
        grid=(_N // _NODE_BLK,),
        in_specs=[_BS_N64, _BS_N1, _full((1, 64)), _full((1, 64)),
                  _full((64, 64))],
        out_specs=_BS_AGG,
        out_shape=_Y_SHAPE,
    )(x_emb, cn, g, b, ws)


def _node_mid(agg, S, x_prev, cn, wsh, sw, sb, g, b, ws, first):
    return pl.pallas_call(
        functools.partial(_node_mid_body, first=first),
        grid=(_N // _NODE_BLK,),
        in_specs=[_BS_AGG, _BS_N16, _BS_N64, _BS_N1, _full((16, 64)),
                  _full((1, 1)), _full((1, 1)), _full((1, 64)),
                  _full((1, 64)), _full((64, 64))],
        out_specs=[_BS_AGG, _BS_N64],
        out_shape=[_Y_SHAPE, _X_SHAPE],
    )(agg, S, x_prev, cn, wsh, sw, sb, g, b, ws)


def _node_last(agg, S, x_prev, cn, wsh, sw, sb, wout, bout, gain):
    return pl.pallas_call(
        _node_last_body,
        grid=(_N // _NODE_BLK,),
        in_specs=[_BS_AGG, _BS_N16, _BS_N64, _BS_N1, _full((16, 64)),
                  _full((1, 1)), _full((1, 1)), _full((64, 3)),
                  _full((1, 3)), _full((1, 1))],
        out_specs=pl.BlockSpec((_NODE_BLK, 3), lambda i: (i, 0)),
        out_shape=jax.ShapeDtypeStruct((_N, 3), jnp.float32),
    )(agg, S, x_prev, cn, wsh, sw, sb, wout, bout, gain)


def _as11(x):
    return jnp.asarray(x, jnp.float32).reshape(1, 1)


def kernel(pos, c_noise, params, edge_index, bond_mask, atom_types):
    src = jnp.pad(edge_index[0].astype(jnp.int32), (0, _EP - _E))
    dst = jnp.pad(edge_index[1].astype(jnp.int32), (0, _EP - _E))

    pos16 = jnp.pad(pos, ((0, 0), (0, 13)))
    offs = (jnp.arange(4, dtype=jnp.int32) * _N)[:, None]
    srcs4 = (src[None, :] + offs).reshape(4, _ER, _W)
    dst3 = dst.reshape(_ER, _W)
    zeros16 = jnp.zeros((_N, 16), jnp.float32)
    bm_pad = jnp.pad(bond_mask.astype(jnp.int32), (0, _EP - _E))[:, None]

    ev3 = _sc_edge_vec(pos16, srcs4, dst3)

    wpack = _pack_edge_weights(params)
    sh, rad0, rad1, rad2 = _edge_dense(ev3.reshape(_EP, 16), bm_pad, wpack)
    radials = (rad0, rad1, rad2)

    S2 = _sc_scatter_sh(sh.reshape(_ER, _W, 16), dst3, zeros16)
    S = S2[0] + S2[1]

    S, zeros16 = lax.optimization_barrier((S, zeros16))

    cn = c_noise[:, None]
    x_emb = params["emb_atom"][atom_types]

    def wsh_pad(i):
        return jnp.concatenate(
            [params["conv%d" % i]["Wsh"], jnp.zeros((12, 64), jnp.float32)],
            0)

    def conv_sc(y, i):
        return _sc_conv(y.reshape(4 * _N, 16),
                        radials[i].reshape(4, _ER, _W, 16),
                        srcs4, dst3, zeros16)

    sc0 = params["scale0"]
    y = _node_first(x_emb, cn, sc0["g"][None, :], sc0["b"][None, :],
                    params["conv0"]["Wself"])
    agg = conv_sc(y, 0)

    x_prev = jnp.zeros((_N, 64), jnp.float32)
    one = _as11(1.0)
    for i in (1, 2):
        sc = params["scale%d" % i]
        sk = params["skip%d" % (i - 1)] if i > 1 else None
        y, x_prev = _node_mid(
            agg, S, x_prev, cn, wsh_pad(i - 1),
            _as11(sk["w"]) if i > 1 else one,
            _as11(sk["b"]) if i > 1 else one,
            sc["g"][None, :], sc["b"][None, :],
            params["conv%d" % i]["Wself"],
            first=(i == 1),
        )
        agg = conv_sc(y, i)

    sk2 = params["skip2"]
    out = _node_last(agg, S, x_prev, cn, wsh_pad(2), _as11(sk2["w"]),
                     _as11(sk2["b"]), params["Wout"],
                     params["bout"][None, :], _as11(params["gain"]))
    return out
